# Initial kernel scaffold; baseline (speedup 1.0000x reference)
import jax
import jax.numpy as jnp
from jax.experimental import pallas as pl


def kernel(input, index, value):
    raise NotImplementedError("write your pallas kernel here")


# speedup vs baseline: 1.0905x; 1.0905x over previous
import functools

import jax
import jax.numpy as jnp
from jax import lax
from jax.experimental import pallas as pl
from jax.experimental.pallas import tpu as pltpu
from jax.experimental.pallas import tpu_sc as plsc

N_ROWS = 100000
N_UPD = 16384
L = 16
NW = 32
RPW = 3128
M_VECS = (RPW + L - 1) // L
M_PAD = M_VECS * L
GCH = 128
W_CAP = M_PAD + 2 * GCH
CPY = 128
SCAN_VECS = N_UPD // L


def _impl(x32, idx32, v32):
    width = x32.shape[1]
    dt = jnp.int32
    mesh = plsc.VectorSubcoreMesh(
        core_axis_name="c", subcore_axis_name="s", num_cores=2,
        num_subcores=16)

    @functools.partial(
        pl.kernel,
        out_type=jax.ShapeDtypeStruct((N_ROWS, width), dt),
        mesh=mesh,
        compiler_params=pltpu.CompilerParams(needs_layout_passes=False),
        scratch_types=[
            pltpu.VMEM((N_UPD,), jnp.int32),
            pltpu.VMEM((M_PAD,), jnp.int32),
            pltpu.VMEM((W_CAP,), jnp.int32),
            pltpu.VMEM((W_CAP,), jnp.int32),
            pltpu.VMEM((GCH, width), dt),
            pltpu.VMEM((GCH,), jnp.int32),
            pltpu.VMEM((2, CPY, width), dt),
            pltpu.SemaphoreType.DMA((2,)),
            pltpu.SemaphoreType.DMA((2,)),
            pltpu.SemaphoreType.DMA,
            pltpu.SemaphoreType.DMA,
        ],
    )
    def run(x_hbm, idx_hbm, v_hbm, out_hbm, idx_v, m_v, wpos, wrow, gbuf,
            sidx, cb, isem, osem, gsem, ssem):
        i32 = jnp.int32
        wid = lax.axis_index("s") * i32(2) + lax.axis_index("c")
        lo = pl.multiple_of(wid * i32(RPW), 8)
        rows_mine = jnp.minimum(i32(RPW), i32(N_ROWS) - lo)
        lanes = lax.iota(jnp.int32, L)

        pltpu.sync_copy(idx_hbm, idx_v)

        def init_m(v, carry):
            m_v[pl.ds(v * i32(L), L)] = jnp.full((L,), -1, jnp.int32)
            return carry
        lax.fori_loop(i32(0), i32(M_VECS), init_m, i32(0))

        def scan_step(v, carry):
            base = v * i32(L)
            ivec = idx_v[pl.ds(base, L)]
            r = ivec - lo
            inm = (r >= 0) & (r < rows_mine)
            cnt = plsc.all_reduce_population_count(inm)[0]
            pos = base + lanes
            rc = jnp.where(inm, r, i32(0))

            @pl.when(cnt == 1)
            def _():
                plsc.store_scatter(m_v, [rc], pos, mask=inm)

            @pl.when(cnt > 1)
            def _():
                rm = jnp.where(inm, r, i32(-1))
                lose = jnp.zeros((L,), jnp.bool_)
                for k in range(1, L):
                    shifted = rm.at[jnp.minimum(lanes + i32(k), i32(L - 1))].get(
                        mode="promise_in_bounds")
                    valid = lanes < i32(L - k)
                    lose = lose | (valid & (shifted == rm))
                keep = inm & jnp.logical_not(lose)
                plsc.store_scatter(m_v, [rc], pos, mask=keep)

            return carry
        lax.fori_loop(i32(0), i32(SCAN_VECS), scan_step, i32(0))

        nt = (rows_mine + i32(CPY - 1)) // i32(CPY)
        last_rel = rows_mine - i32(CPY)

        def chunk_abs(c):
            rel = jnp.minimum(c * i32(CPY), last_rel)
            return pl.multiple_of(lo + rel, 8)

        def in_desc(c, b):
            return pltpu.make_async_copy(
                x_hbm.at[pl.ds(chunk_abs(c), CPY)], cb.at[b], isem.at[b])

        def out_desc(c, b):
            return pltpu.make_async_copy(
                cb.at[b], out_hbm.at[pl.ds(chunk_abs(c), CPY)], osem.at[b])

        in_desc(i32(0), i32(0)).start()

        def copy_body(c, carry):
            b = c & i32(1)
            bn = (c + i32(1)) & i32(1)

            @pl.when(c + i32(1) < nt)
            def _():
                @pl.when(c >= i32(1))
                def _():
                    out_desc(c - i32(1), bn).wait()
                in_desc(c + i32(1), bn).start()

            in_desc(c, b).wait()
            out_desc(c, b).start()
            return carry
        lax.fori_loop(i32(0), nt, copy_body, i32(0))

        out_desc(nt - i32(2), nt & i32(1)).wait()
        out_desc(nt - i32(1), (nt - i32(1)) & i32(1)).wait()

        def compress_step(v, carry):
            wcount, lastrow, lastpos = carry
            m = m_v[pl.ds(v * i32(L), L)]
            msk = m >= i32(0)
            cnt = plsc.all_reduce_population_count(msk)[0]
            rows = lo + v * i32(L) + lanes
            plsc.store_compressed(wpos.at[pl.ds(wcount, L)], m, mask=msk)
            plsc.store_compressed(wrow.at[pl.ds(wcount, L)], rows, mask=msk)
            lr = jnp.max(jnp.where(msk, rows, i32(-1)))
            lp = jnp.max(jnp.where(msk & (rows == lr), m, i32(-1)))
            lastrow = jnp.where(cnt > i32(0), lr, lastrow)
            lastpos = jnp.where(cnt > i32(0), lp, lastpos)
            return (wcount + cnt, lastrow, lastpos)

        wcount, lastrow, lastpos = lax.fori_loop(
            i32(0), i32(M_VECS), compress_step,
            (jnp.int32(0), jnp.int32(0), jnp.int32(0)))

        @pl.when(wcount > i32(0))
        def _():
            prow = jnp.full((L,), lastrow, jnp.int32)
            ppos = jnp.full((L,), lastpos, jnp.int32)
            for j in range(GCH // L):
                wrow[pl.ds(wcount + i32(j * L), L)] = prow
                wpos[pl.ds(wcount + i32(j * L), L)] = ppos

        nch = (wcount + i32(GCH - 1)) // i32(GCH)

        def win_step(c, carry):
            off = c * i32(GCH)
            pltpu.async_copy(
                v_hbm.at[wpos.at[pl.ds(off, GCH)]], gbuf, gsem).wait()
            for j in range(GCH // L):
                sidx[pl.ds(j * L, L)] = wrow[pl.ds(off + i32(j * L), L)]
            pltpu.async_copy(gbuf, out_hbm.at[sidx], ssem).wait()
            return carry
        lax.fori_loop(i32(0), nch, win_step, i32(0))

    return run(x32, idx32, v32)


def kernel(input, index, value):
    idx32 = index.astype(jnp.int32)
    if input.dtype == jnp.int64:
        x32 = lax.bitcast_convert_type(input, jnp.int32).reshape(N_ROWS, 256)
        v32 = lax.bitcast_convert_type(value, jnp.int32).reshape(N_UPD, 256)
        out32 = _impl(x32, idx32, v32)
        return lax.bitcast_convert_type(
            out32.reshape(N_ROWS, 128, 2), jnp.int64)
    return _impl(input, idx32, value)


# speedup vs baseline: 1.8738x; 1.7182x over previous
import functools

import jax
import jax.numpy as jnp
from jax import lax
from jax.experimental import pallas as pl
from jax.experimental.pallas import tpu as pltpu
from jax.experimental.pallas import tpu_sc as plsc

N_ROWS = 100000
N_UPD = 16384
WIDTH = 128
L = 16
NW = 32
RPW = 3128
M_VECS = (RPW + L - 1) // L
M_PAD = M_VECS * L
GCH = 128
W_CAP = M_PAD + 2 * GCH
CPY = 128
SCAN_VECS = N_UPD // L


def _impl(planes_x, idx32, planes_v):
    np_ = len(planes_x)
    mesh = plsc.VectorSubcoreMesh(
        core_axis_name="c", subcore_axis_name="s", num_cores=2,
        num_subcores=16)

    scratch = [
        pltpu.VMEM((N_UPD,), jnp.int32),
        pltpu.VMEM((M_PAD,), jnp.int32),
        pltpu.VMEM((W_CAP,), jnp.int32),
        pltpu.VMEM((W_CAP,), jnp.int32),
        pltpu.VMEM((GCH,), jnp.int32),
    ]
    for _ in range(np_):
        scratch.append(pltpu.VMEM((GCH, WIDTH), jnp.int32))
        scratch.append(pltpu.VMEM((2, CPY, WIDTH), jnp.int32))
        scratch.append(pltpu.SemaphoreType.DMA((2,)))
        scratch.append(pltpu.SemaphoreType.DMA((2,)))
        scratch.append(pltpu.SemaphoreType.DMA)
        scratch.append(pltpu.SemaphoreType.DMA)

    @functools.partial(
        pl.kernel,
        out_type=tuple(jax.ShapeDtypeStruct((N_ROWS, WIDTH), jnp.int32)
                       for _ in range(np_)),
        mesh=mesh,
        compiler_params=pltpu.CompilerParams(needs_layout_passes=False),
        scratch_types=scratch,
    )
    def run(*refs):
        xs = refs[:np_]
        idx_hbm = refs[np_]
        vs = refs[np_ + 1:2 * np_ + 1]
        outs = refs[2 * np_ + 1:3 * np_ + 1]
        idx_v, m_v, wpos, wrow, sidx = refs[3 * np_ + 1:3 * np_ + 6]
        pp = refs[3 * np_ + 6:]
        gbufs = [pp[6 * p + 0] for p in range(np_)]
        cbs = [pp[6 * p + 1] for p in range(np_)]
        isems = [pp[6 * p + 2] for p in range(np_)]
        osems = [pp[6 * p + 3] for p in range(np_)]
        gsems = [pp[6 * p + 4] for p in range(np_)]
        ssems = [pp[6 * p + 5] for p in range(np_)]

        i32 = jnp.int32
        wid = lax.axis_index("s") * i32(2) + lax.axis_index("c")
        lo = pl.multiple_of(wid * i32(RPW), 8)
        rows_mine = jnp.minimum(i32(RPW), i32(N_ROWS) - lo)
        lanes = lax.iota(jnp.int32, L)

        pltpu.sync_copy(idx_hbm, idx_v)

        def init_m(v, carry):
            m_v[pl.ds(v * i32(L), L)] = jnp.full((L,), -1, jnp.int32)
            return carry
        lax.fori_loop(i32(0), i32(M_VECS), init_m, i32(0))

        def scan_step(v, carry):
            base = v * i32(L)
            ivec = idx_v[pl.ds(base, L)]
            r = ivec - lo
            inm = (r >= 0) & (r < rows_mine)
            cnt = plsc.all_reduce_population_count(inm)[0]
            pos = base + lanes
            rc = jnp.where(inm, r, i32(0))

            @pl.when(cnt == 1)
            def _():
                plsc.store_scatter(m_v, [rc], pos, mask=inm)

            @pl.when(cnt > 1)
            def _():
                rm = jnp.where(inm, r, i32(-1))
                lose = jnp.zeros((L,), jnp.bool_)
                for k in range(1, L):
                    shifted = rm.at[jnp.minimum(lanes + i32(k), i32(L - 1))].get(
                        mode="promise_in_bounds")
                    valid = lanes < i32(L - k)
                    lose = lose | (valid & (shifted == rm))
                keep = inm & jnp.logical_not(lose)
                plsc.store_scatter(m_v, [rc], pos, mask=keep)

            return carry
        lax.fori_loop(i32(0), i32(SCAN_VECS), scan_step, i32(0))

        nt = (rows_mine + i32(CPY - 1)) // i32(CPY)
        last_rel = rows_mine - i32(CPY)

        def chunk_abs(c):
            rel = jnp.minimum(c * i32(CPY), last_rel)
            return pl.multiple_of(lo + rel, 8)

        def in_descs(c, b):
            return [pltpu.make_async_copy(
                xs[p].at[pl.ds(chunk_abs(c), CPY)], cbs[p].at[b],
                isems[p].at[b]) for p in range(np_)]

        def out_descs(c, b):
            return [pltpu.make_async_copy(
                cbs[p].at[b], outs[p].at[pl.ds(chunk_abs(c), CPY)],
                osems[p].at[b]) for p in range(np_)]

        for d in in_descs(i32(0), i32(0)):
            d.start()

        def copy_body(c, carry):
            b = c & i32(1)
            bn = (c + i32(1)) & i32(1)

            @pl.when(c + i32(1) < nt)
            def _():
                @pl.when(c >= i32(1))
                def _():
                    for d in out_descs(c - i32(1), bn):
                        d.wait()
                for d in in_descs(c + i32(1), bn):
                    d.start()

            for d in in_descs(c, b):
                d.wait()
            for d in out_descs(c, b):
                d.start()
            return carry
        lax.fori_loop(i32(0), nt, copy_body, i32(0))

        for d in out_descs(nt - i32(2), nt & i32(1)):
            d.wait()
        for d in out_descs(nt - i32(1), (nt - i32(1)) & i32(1)):
            d.wait()

        def compress_step(v, carry):
            wcount, lastrow, lastpos = carry
            m = m_v[pl.ds(v * i32(L), L)]
            msk = m >= i32(0)
            cnt = plsc.all_reduce_population_count(msk)[0]
            rows = lo + v * i32(L) + lanes
            plsc.store_compressed(wpos.at[pl.ds(wcount, L)], m, mask=msk)
            plsc.store_compressed(wrow.at[pl.ds(wcount, L)], rows, mask=msk)
            lr = jnp.max(jnp.where(msk, rows, i32(-1)))
            lp = jnp.max(jnp.where(msk & (rows == lr), m, i32(-1)))
            lastrow = jnp.where(cnt > i32(0), lr, lastrow)
            lastpos = jnp.where(cnt > i32(0), lp, lastpos)
            return (wcount + cnt, lastrow, lastpos)

        wcount, lastrow, lastpos = lax.fori_loop(
            i32(0), i32(M_VECS), compress_step,
            (jnp.int32(0), jnp.int32(0), jnp.int32(0)))

        @pl.when(wcount > i32(0))
        def _():
            prow = jnp.full((L,), lastrow, jnp.int32)
            ppos = jnp.full((L,), lastpos, jnp.int32)
            for j in range(GCH // L):
                wrow[pl.ds(wcount + i32(j * L), L)] = prow
                wpos[pl.ds(wcount + i32(j * L), L)] = ppos

        nch = (wcount + i32(GCH - 1)) // i32(GCH)

        def win_step(c, carry):
            off = c * i32(GCH)
            gds = [pltpu.make_async_copy(
                vs[p].at[wpos.at[pl.ds(off, GCH)]], gbufs[p], gsems[p])
                for p in range(np_)]
            for d in gds:
                d.start()
            for j in range(GCH // L):
                sidx[pl.ds(j * L, L)] = wrow[pl.ds(off + i32(j * L), L)]
            for d in gds:
                d.wait()
            sds = [pltpu.make_async_copy(
                gbufs[p], outs[p].at[sidx], ssems[p]) for p in range(np_)]
            for d in sds:
                d.start()
            for d in sds:
                d.wait()
            return carry
        lax.fori_loop(i32(0), nch, win_step, i32(0))

    return run(*planes_x, idx32, *planes_v)


def kernel(input, index, value):
    idx32 = index.astype(jnp.int32)
    if input.dtype == jnp.int64:
        mask = jnp.int64(0xFFFFFFFF)

        def split(a):
            lo_p = lax.bitcast_convert_type(
                lax.convert_element_type(a & mask, jnp.uint32), jnp.int32)
            hi_p = lax.convert_element_type(a >> jnp.int64(32), jnp.int32)
            return lo_p, hi_p

        xlo, xhi = split(input)
        vlo, vhi = split(value)
        outlo, outhi = _impl([xlo, xhi], idx32, [vlo, vhi])
        lo64 = lax.convert_element_type(
            lax.bitcast_convert_type(outlo, jnp.uint32), jnp.int64)
        hi64 = lax.convert_element_type(outhi, jnp.int64) << jnp.int64(32)
        return hi64 | lo64
    out, = _impl([input], idx32, [value])
    return out
